# baseline (speedup 1.0000x reference)
import jax
import jax.numpy as jnp
from jax.experimental import pallas as pl

_BLK = 1024


def _body(x_ref, emb_ref, w4_ref, b_ref, o_ref):
    xb = x_ref[...]
    e0 = emb_ref[0, 0]
    e1 = emb_ref[0, 1]
    col = jax.lax.broadcasted_iota(jnp.int32, (1, 4), 1)
    is_idx_col = (col == 1) | (col == 3)
    e_val = jnp.where(xb >= 1.0, e1, e0)
    e4 = jnp.where(is_idx_col, e_val, xb)
    o_ref[...] = (
        jax.lax.dot_general(
            e4, w4_ref[...],
            dimension_numbers=(((1,), (0,)), ((), ())),
            preferred_element_type=jnp.float32,
        )
        + b_ref[...]
    )


@jax.jit
def _run(x4, emb_row, w4, b2):
    H = x4.shape[0]
    grid = (H // _BLK,)
    return pl.pallas_call(
        _body,
        grid=grid,
        in_specs=[
            pl.BlockSpec((_BLK, 4), lambda i: (i, 0)),
            pl.BlockSpec((1, 2), lambda i: (0, 0)),
            pl.BlockSpec((4, 128), lambda i: (0, 0)),
            pl.BlockSpec((1, 128), lambda i: (0, 0)),
        ],
        out_specs=pl.BlockSpec((_BLK, 128), lambda i: (i, 0)),
        out_shape=jax.ShapeDtypeStruct((H, 128), jnp.float32),
    )(x4, emb_row, w4, b2)


def kernel(x, emb16, fc1_w, fc1_b):
    B, _ = x.shape
    N = fc1_w.shape[0]
    x4 = x.reshape(B // 2, 4)
    emb_row = emb16.reshape(1, 2)
    w0 = fc1_w[:, 0]
    w1 = fc1_w[:, 1]
    z = jnp.zeros((N,), jnp.float32)
    w4 = jnp.stack([
        jnp.concatenate([w0, z]),
        jnp.concatenate([w1, z]),
        jnp.concatenate([z, w0]),
        jnp.concatenate([z, w1]),
    ])
    b2 = jnp.tile(fc1_b, 2).reshape(1, 2 * N)
    out2 = _run(x4, emb_row, w4, b2)
    return out2.reshape(B, N)


# speedup vs baseline: 4.7140x; 4.7140x over previous
import jax
import jax.numpy as jnp
from jax.experimental import pallas as pl

_LBLK = 2048


def _body(xt_ref, emb_ref, w8_ref, o_ref):
    x0 = xt_ref[0:1, :]
    x1 = xt_ref[1:2, :]
    e0 = emb_ref[0, 0]
    e1 = emb_ref[0, 1]
    e = jnp.where(x1 >= 1.0, e1, e0)
    one = jnp.ones_like(x0)
    zero = jnp.zeros((5, x0.shape[1]), jnp.float32)
    m = jnp.concatenate([x0, e, one, zero], axis=0)
    o_ref[...] = jax.lax.dot_general(
        w8_ref[...], m,
        dimension_numbers=(((1,), (0,)), ((), ())),
        preferred_element_type=jnp.float32,
    )


@jax.jit
def _run(xt, emb_row, w8):
    B = xt.shape[1]
    N = w8.shape[0]
    grid = (B // _LBLK,)
    return pl.pallas_call(
        _body,
        grid=grid,
        in_specs=[
            pl.BlockSpec((2, _LBLK), lambda i: (0, i)),
            pl.BlockSpec((1, 2), lambda i: (0, 0)),
            pl.BlockSpec((N, 8), lambda i: (0, 0)),
        ],
        out_specs=pl.BlockSpec((N, _LBLK), lambda i: (0, i)),
        out_shape=jax.ShapeDtypeStruct((N, B), jnp.float32),
    )(xt, emb_row, w8)


def kernel(x, emb16, fc1_w, fc1_b):
    N = fc1_w.shape[0]
    xt = x.T
    emb_row = emb16.reshape(1, 2)
    w8 = jnp.concatenate(
        [fc1_w, fc1_b.reshape(N, 1), jnp.zeros((N, 5), jnp.float32)], axis=1
    )
    out_t = _run(xt, emb_row, w8)
    return out_t.T


# speedup vs baseline: 6.1357x; 1.3016x over previous
import jax
import jax.numpy as jnp
from jax.experimental import pallas as pl

_LBLK = 4096


def _body(xt_ref, emb_ref, w8_ref, o_ref):
    x0 = xt_ref[0:1, :]
    x1 = xt_ref[1:2, :]
    e0 = emb_ref[0, 0]
    e1 = emb_ref[0, 1]
    e = jnp.where(x1 >= 1.0, e1, e0)
    one = jnp.ones_like(x0)
    zero = jnp.zeros((5, x0.shape[1]), jnp.float32)
    m = jnp.concatenate([x0, e, one, zero], axis=0)
    o_ref[...] = jax.lax.dot_general(
        w8_ref[...], m,
        dimension_numbers=(((1,), (0,)), ((), ())),
        preferred_element_type=jnp.float32,
    )


@jax.jit
def _run(xt, emb_row, w8):
    B = xt.shape[1]
    N = w8.shape[0]
    grid = (B // _LBLK,)
    return pl.pallas_call(
        _body,
        grid=grid,
        in_specs=[
            pl.BlockSpec((2, _LBLK), lambda i: (0, i)),
            pl.BlockSpec((1, 2), lambda i: (0, 0)),
            pl.BlockSpec((N, 8), lambda i: (0, 0)),
        ],
        out_specs=pl.BlockSpec((N, _LBLK), lambda i: (0, i)),
        out_shape=jax.ShapeDtypeStruct((N, B), jnp.float32),
    )(xt, emb_row, w8)


def kernel(x, emb16, fc1_w, fc1_b):
    N = fc1_w.shape[0]
    xt = x.T
    emb_row = emb16.reshape(1, 2)
    w8 = jnp.concatenate(
        [fc1_w, fc1_b.reshape(N, 1), jnp.zeros((N, 5), jnp.float32)], axis=1
    )
    out_t = _run(xt, emb_row, w8)
    return out_t.T


# speedup vs baseline: 7.5413x; 1.2291x over previous
import jax
import jax.numpy as jnp
from jax.experimental import pallas as pl

_LBLK = 8192


def _body(xt_ref, emb_ref, w8_ref, o_ref):
    x0 = xt_ref[0:1, :]
    x1 = xt_ref[1:2, :]
    e0 = emb_ref[0, 0]
    e1 = emb_ref[0, 1]
    e = jnp.where(x1 >= 1.0, e1, e0)
    one = jnp.ones_like(x0)
    zero = jnp.zeros((5, x0.shape[1]), jnp.float32)
    m = jnp.concatenate([x0, e, one, zero], axis=0)
    o_ref[...] = jax.lax.dot_general(
        w8_ref[...], m,
        dimension_numbers=(((1,), (0,)), ((), ())),
        preferred_element_type=jnp.float32,
    )


@jax.jit
def _run(xt, emb_row, w8):
    B = xt.shape[1]
    N = w8.shape[0]
    grid = (B // _LBLK,)
    return pl.pallas_call(
        _body,
        grid=grid,
        in_specs=[
            pl.BlockSpec((2, _LBLK), lambda i: (0, i)),
            pl.BlockSpec((1, 2), lambda i: (0, 0)),
            pl.BlockSpec((N, 8), lambda i: (0, 0)),
        ],
        out_specs=pl.BlockSpec((N, _LBLK), lambda i: (0, i)),
        out_shape=jax.ShapeDtypeStruct((N, B), jnp.float32),
    )(xt, emb_row, w8)


def kernel(x, emb16, fc1_w, fc1_b):
    N = fc1_w.shape[0]
    xt = x.T
    emb_row = emb16.reshape(1, 2)
    w8 = jnp.concatenate(
        [fc1_w, fc1_b.reshape(N, 1), jnp.zeros((N, 5), jnp.float32)], axis=1
    )
    out_t = _run(xt, emb_row, w8)
    return out_t.T
